# baseline (speedup 1.0000x reference)
import jax
import jax.numpy as jnp
from jax.experimental import pallas as pl
from jax.experimental.pallas import tpu as pltpu

MODEL_DIM = 2048
NUM_EXPERTS = 16
TOP_K = 2
TILE = 512


def _gate_body(x_ref, w_ref, logits_ref, wts_ref, idx_ref):
    x = x_ref[...]
    w = w_ref[...]
    logits_t = jax.lax.dot_general(
        w, x, (((1,), (1,)), ((), ())),
        preferred_element_type=jnp.float32)
    logits_ref[...] = logits_t

    expert = jax.lax.broadcasted_iota(jnp.int32, logits_t.shape, 0)
    m1 = jnp.max(logits_t, axis=0, keepdims=True)
    i1 = jnp.min(jnp.where(logits_t == m1, expert, NUM_EXPERTS), axis=0,
                 keepdims=True)
    masked = jnp.where(expert == i1, -jnp.inf, logits_t)
    m2 = jnp.max(masked, axis=0, keepdims=True)
    i2 = jnp.min(jnp.where(masked == m2, expert, NUM_EXPERTS), axis=0,
                 keepdims=True)
    e = jnp.exp(m2 - m1)
    w1 = 1.0 / (1.0 + e)
    w2 = e * w1
    wts_ref[...] = jnp.concatenate([w1, w2], axis=0)
    idx_ref[...] = jnp.concatenate([i1, i2], axis=0)


@jax.jit
def kernel(x, W):
    n_tokens = x.shape[0]
    logits_t, wts_t, idx_t = pl.pallas_call(
        _gate_body,
        grid=(n_tokens // TILE,),
        in_specs=[
            pl.BlockSpec((TILE, MODEL_DIM), lambda i: (i, 0)),
            pl.BlockSpec((NUM_EXPERTS, MODEL_DIM), lambda i: (0, 0)),
        ],
        out_specs=[
            pl.BlockSpec((NUM_EXPERTS, TILE), lambda i: (0, i)),
            pl.BlockSpec((TOP_K, TILE), lambda i: (0, i)),
            pl.BlockSpec((TOP_K, TILE), lambda i: (0, i)),
        ],
        out_shape=[
            jax.ShapeDtypeStruct((NUM_EXPERTS, n_tokens), jnp.float32),
            jax.ShapeDtypeStruct((TOP_K, n_tokens), jnp.float32),
            jax.ShapeDtypeStruct((TOP_K, n_tokens), jnp.int32),
        ],
        compiler_params=pltpu.CompilerParams(
            dimension_semantics=("arbitrary",),
            vmem_limit_bytes=50 * 1024 * 1024,
        ),
    )(x, W)
    return wts_t.T, idx_t.T, logits_t.T
